# baseline (speedup 1.0000x reference)
import functools

import jax
import jax.numpy as jnp
from jax import lax
from jax.experimental import pallas as pl
from jax.experimental.pallas import tpu as pltpu
from jax.experimental.pallas import tpu_sc as plsc

D = 128
NC = 2
NS = 16
NW = NC * NS
CH = 128
NBUF = 4


def _sc_gather(table, ids_t):
  n_ch, b = ids_t.shape
  total = n_ch * b
  mesh = plsc.VectorSubcoreMesh(
      core_axis_name="c", subcore_axis_name="s", num_cores=NC, num_subcores=NS
  )

  @functools.partial(
      pl.kernel,
      out_type=jax.ShapeDtypeStruct((total, D), jnp.float32),
      mesh=mesh,
      scratch_types=[
          pltpu.VMEM((n_ch, CH), jnp.int32),
          [pltpu.VMEM((CH, D), jnp.float32) for _ in range(NBUF)],
          [pltpu.SemaphoreType.DMA for _ in range(NBUF)],
      ],
  )
  def k(table_hbm, ids_hbm, out_hbm, idx_v, rows, sems):
    wid = lax.axis_index("s") * NC + lax.axis_index("c")
    col0 = wid * CH
    pltpu.sync_copy(ids_hbm.at[pl.ds(0, n_ch), pl.ds(col0, CH)], idx_v)
    for j in range(min(NBUF, n_ch)):
      pltpu.async_copy(table_hbm.at[idx_v.at[j]], rows[j], sems[j])

    def quad(q, _):
      j0 = NBUF * q
      for t in range(NBUF):
        j = j0 + t
        pltpu.make_async_copy(
            table_hbm.at[idx_v.at[j]], rows[t], sems[t]).wait()
        pltpu.sync_copy(rows[t], out_hbm.at[pl.ds(j * b + col0, CH)])

        @pl.when(j + NBUF < n_ch)
        def _():
          pltpu.async_copy(table_hbm.at[idx_v.at[j + NBUF]], rows[t], sems[t])

      return ()

    lax.fori_loop(0, n_ch // NBUF, quad, ())

    for t in range(n_ch % NBUF):
      j = (n_ch // NBUF) * NBUF + t
      pltpu.make_async_copy(
          table_hbm.at[idx_v.at[j]], rows[t], sems[t]).wait()
      pltpu.sync_copy(rows[t], out_hbm.at[pl.ds(j * b + col0, CH)])

  return k(table, ids_t)


def _tc_main(x, g, wa, ba, wt, bt, rows_per_tile, tile0, n_tiles, l_seg):
  r = rows_per_tile
  b = (n_tiles * r) // l_seg

  def body(x_ref, g_ref, wa_ref, ba_ref, wt_ref, bt_ref, out_ref, m_s):
    @pl.when(pl.program_id(0) == 0)
    def _():
      m_s[...] = lax.dot_general(
          wa_ref[...], wt_ref[...], (((0,), (0,)), ((), ())),
          preferred_element_type=jnp.float32)

    xv = x_ref[...]
    gv = g_ref[...]
    v = jnp.dot(ba_ref[...], wt_ref[...], preferred_element_type=jnp.float32)
    u = jnp.dot(bt_ref[...], wa_ref[...], preferred_element_type=jnp.float32)
    c = jnp.sum(ba_ref[...] * bt_ref[...])
    a = jnp.dot(xv, m_s[...], preferred_element_type=jnp.float32) + v
    ones = jnp.ones((1, D), dtype=jnp.float32)
    res = lax.dot_general(
        ones, a * gv, (((1,), (1,)), ((), ())),
        preferred_element_type=jnp.float32)
    z = lax.dot_general(
        u, xv, (((1,), (1,)), ((), ())),
        preferred_element_type=jnp.float32)
    if r >= b:
      out_ref[...] = (res + z + c).reshape(r // b, 1, b)
    else:
      out_ref[...] = (res + z + c).reshape(1, 1, r)

  out = pl.pallas_call(
      body,
      grid=(n_tiles,),
      in_specs=[
          pl.BlockSpec((r, D), lambda i: (tile0 + i, 0)),
          pl.BlockSpec((r, D), lambda i: (i, 0)),
          pl.BlockSpec((D, D), lambda i: (0, 0)),
          pl.BlockSpec((1, D), lambda i: (0, 0)),
          pl.BlockSpec((D, D), lambda i: (0, 0)),
          pl.BlockSpec((1, D), lambda i: (0, 0)),
      ],
      out_specs=(
          pl.BlockSpec((r // b, 1, b), lambda i: (i, 0, 0))
          if r >= b else
          pl.BlockSpec((1, 1, r), lambda i: (i // (b // r), 0, i % (b // r)))
      ),
      out_shape=jax.ShapeDtypeStruct((l_seg, 1, b), jnp.float32),
      scratch_shapes=[pltpu.VMEM((D, D), jnp.float32)],
  )(x, g, wa, ba, wt, bt)
  return out


def kernel(actor_emb, topic_ids, Wa, ba, table, Wt, bt, scale):
  b, l, d = actor_emb.shape
  bl = b * l

  wa_s = Wa * scale
  ba_s = (ba * scale).reshape(1, d)

  ids_t = topic_ids.T.astype(jnp.int32)
  x = actor_emb.transpose(1, 0, 2).reshape(bl, d)

  n_seg = 5
  l_seg = l // n_seg
  r = 20480
  nt_seg = l_seg * b // r
  bt_r = bt.reshape(1, d)
  outs = []
  for s in range(n_seg):
    ids_seg = lax.slice_in_dim(ids_t, s * l_seg, (s + 1) * l_seg, axis=0)
    g_seg = _sc_gather(table, ids_seg)
    outs.append(_tc_main(x, g_seg, wa_s, ba_s, Wt, bt_r, r,
                         s * nt_seg, nt_seg, l_seg))
  out = jnp.concatenate(outs, axis=0)
  return out.reshape(l, b).T


# speedup vs baseline: 1.0690x; 1.0690x over previous
import functools

import jax
import jax.numpy as jnp
from jax import lax
from jax.experimental import pallas as pl
from jax.experimental.pallas import tpu as pltpu
from jax.experimental.pallas import tpu_sc as plsc

D = 128
NC = 2
NS = 16
NW = NC * NS
CH = 128
NBUF = 4


def _sc_gather(table, ids_t):
  n_ch, b = ids_t.shape
  n_pairs = n_ch // 2
  total2 = n_pairs * b
  mesh = plsc.VectorSubcoreMesh(
      core_axis_name="c", subcore_axis_name="s", num_cores=NC, num_subcores=NS
  )

  def _pack_pair(rows_a, rows_b, pk):
    def rowbody(j, _):
      for grp in range(D // 16):
        cc = grp * 16
        ua = lax.bitcast_convert_type(rows_a[j, pl.ds(cc, 16)], jnp.int32)
        ub = lax.bitcast_convert_type(rows_b[j, pl.ds(cc, 16)], jnp.int32)
        ra = lax.shift_right_logical(
            ua + 32767 + ((ua >> 16) & 1), 16)
        rb = (ub + 32767 + ((ub >> 16) & 1)) & jnp.int32(-65536)
        pk[j, pl.ds(cc, 16)] = ra | rb
      return ()

    lax.fori_loop(0, CH, rowbody, ())

  @functools.partial(
      pl.kernel,
      out_type=jax.ShapeDtypeStruct((total2, D), jnp.int32),
      mesh=mesh,
      scratch_types=[
          pltpu.VMEM((n_ch, CH), jnp.int32),
          [pltpu.VMEM((CH, D), jnp.float32) for _ in range(4)],
          [pltpu.VMEM((CH, D), jnp.int32) for _ in range(2)],
          [pltpu.SemaphoreType.DMA for _ in range(4)],
      ],
  )
  def k(table_hbm, ids_hbm, out_hbm, idx_v, rows, pks, sems):
    wid = lax.axis_index("s") * NC + lax.axis_index("c")
    col0 = wid * CH
    pltpu.sync_copy(ids_hbm.at[pl.ds(0, n_ch), pl.ds(col0, CH)], idx_v)
    for j in range(min(4, n_ch)):
      pltpu.async_copy(table_hbm.at[idx_v.at[j]], rows[j], sems[j])

    def handle_pair(p, b0):
      for t in (b0, b0 + 1):
        pltpu.make_async_copy(
            table_hbm.at[idx_v.at[2 * p + t - b0]], rows[t], sems[t]).wait()
      _pack_pair(rows[b0], rows[b0 + 1], pks[b0 // 2])

      @pl.when(2 * p + 4 < n_ch)
      def _():
        for t in (b0, b0 + 1):
          pltpu.async_copy(
              table_hbm.at[idx_v.at[2 * p + 4 + t - b0]], rows[t], sems[t])

      pltpu.sync_copy(pks[b0 // 2], out_hbm.at[pl.ds(p * b + col0, CH)])

    def duo(q, _):
      handle_pair(2 * q, 0)
      handle_pair(2 * q + 1, 2)
      return ()

    lax.fori_loop(0, n_pairs // 2, duo, ())
    if n_pairs % 2:
      handle_pair(jnp.int32(n_pairs - 1), 0)

  return k(table, ids_t)


def _tc_main(x, g, wa, ba, wt, bt, rows_per_tile, tile0, n_tiles, l_seg):
  r = rows_per_tile
  b = (n_tiles * r) // l_seg

  def body(x_ref, g_ref, wa_ref, ba_ref, wt_ref, bt_ref, out_ref, m_s):
    @pl.when(pl.program_id(0) == 0)
    def _():
      m_s[...] = lax.dot_general(
          wa_ref[...], wt_ref[...], (((0,), (0,)), ((), ())),
          preferred_element_type=jnp.float32)

    xv = x_ref[...]
    gv2 = g_ref[...]
    glo = lax.bitcast_convert_type(gv2 << 16, jnp.float32)
    ghi = lax.bitcast_convert_type(gv2 & jnp.int32(-65536), jnp.float32)
    v = jnp.dot(ba_ref[...], wt_ref[...], preferred_element_type=jnp.float32)
    u = jnp.dot(bt_ref[...], wa_ref[...], preferred_element_type=jnp.float32)
    c = jnp.sum(ba_ref[...] * bt_ref[...])
    a = jnp.dot(xv, m_s[...], preferred_element_type=jnp.float32) + v
    ones = jnp.ones((1, D), dtype=jnp.float32)
    r2 = r // 2
    res_lo = lax.dot_general(
        ones, a[:r2] * glo, (((1,), (1,)), ((), ())),
        preferred_element_type=jnp.float32)
    res_hi = lax.dot_general(
        ones, a[r2:] * ghi, (((1,), (1,)), ((), ())),
        preferred_element_type=jnp.float32)
    z = lax.dot_general(
        u, xv, (((1,), (1,)), ((), ())),
        preferred_element_type=jnp.float32)
    row0 = res_lo + z[:, :r2] + c
    row1 = res_hi + z[:, r2:] + c
    out_ref[...] = jnp.concatenate([row0, row1], axis=0).reshape(2, 1, r2)

  out = pl.pallas_call(
      body,
      grid=(n_tiles,),
      in_specs=[
          pl.BlockSpec((r, D), lambda i: (tile0 + i, 0)),
          pl.BlockSpec((r // 2, D), lambda i: (i, 0)),
          pl.BlockSpec((D, D), lambda i: (0, 0)),
          pl.BlockSpec((1, D), lambda i: (0, 0)),
          pl.BlockSpec((D, D), lambda i: (0, 0)),
          pl.BlockSpec((1, D), lambda i: (0, 0)),
      ],
      out_specs=pl.BlockSpec((r // b, 1, b), lambda i: (i, 0, 0)),
      out_shape=jax.ShapeDtypeStruct((l_seg, 1, b), jnp.float32),
      scratch_shapes=[pltpu.VMEM((D, D), jnp.float32)],
  )(x, g, wa, ba, wt, bt)
  return out


def kernel(actor_emb, topic_ids, Wa, ba, table, Wt, bt, scale):
  b, l, d = actor_emb.shape
  bl = b * l

  wa_s = Wa * scale
  ba_s = (ba * scale).reshape(1, d)

  ids_t = topic_ids.T.astype(jnp.int32)
  x = actor_emb.transpose(1, 0, 2).reshape(bl, d)

  n_seg = 5
  l_seg = l // n_seg
  r = 8192
  nt_seg = l_seg * b // r
  bt_r = bt.reshape(1, d)
  outs = []
  for s in range(n_seg):
    ids_seg = lax.slice_in_dim(ids_t, s * l_seg, (s + 1) * l_seg, axis=0)
    g_seg = _sc_gather(table, ids_seg)
    outs.append(_tc_main(x, g_seg, wa_s, ba_s, Wt, bt_r, r,
                         s * nt_seg, nt_seg, l_seg))
  out = jnp.concatenate(outs, axis=0)
  return out.reshape(l, b).T


# speedup vs baseline: 1.1841x; 1.1076x over previous
import functools

import jax
import jax.numpy as jnp
from jax import lax
from jax.experimental import pallas as pl
from jax.experimental.pallas import tpu as pltpu
from jax.experimental.pallas import tpu_sc as plsc

D = 128
NC = 2
NS = 16
NW = NC * NS
CH = 128
NBUF = 4


def _sc_gather(table, ids_t):
  n_ch, b = ids_t.shape
  n_pairs = n_ch // 2
  total2 = n_pairs * b
  mesh = plsc.VectorSubcoreMesh(
      core_axis_name="c", subcore_axis_name="s", num_cores=NC, num_subcores=NS
  )

  def _pack_pair(rows_a, rows_b, pk):
    def rowbody(j, _):
      for grp in range(D // 16):
        cc = grp * 16
        ua = lax.bitcast_convert_type(rows_a[j, pl.ds(cc, 16)], jnp.int32)
        ub = lax.bitcast_convert_type(rows_b[j, pl.ds(cc, 16)], jnp.int32)
        ra = lax.shift_right_logical(ua + 32768, 16)
        rb = (ub + 32768) & jnp.int32(-65536)
        pk[j, pl.ds(cc, 16)] = ra | rb
      return ()

    lax.fori_loop(0, CH, rowbody, ())

  @functools.partial(
      pl.kernel,
      out_type=jax.ShapeDtypeStruct((total2, D), jnp.int32),
      mesh=mesh,
      scratch_types=[
          pltpu.VMEM((n_ch, CH), jnp.int32),
          [pltpu.VMEM((CH, D), jnp.float32) for _ in range(4)],
          [pltpu.VMEM((CH, D), jnp.int32) for _ in range(2)],
          [pltpu.SemaphoreType.DMA for _ in range(4)],
      ],
  )
  def k(table_hbm, ids_hbm, out_hbm, idx_v, rows, pks, sems):
    wid = lax.axis_index("s") * NC + lax.axis_index("c")
    col0 = wid * CH
    pltpu.sync_copy(ids_hbm.at[pl.ds(0, n_ch), pl.ds(col0, CH)], idx_v)
    for j in range(min(4, n_ch)):
      pltpu.async_copy(table_hbm.at[idx_v.at[j]], rows[j], sems[j])

    def handle_pair(p, b0):
      for t in (b0, b0 + 1):
        pltpu.make_async_copy(
            table_hbm.at[idx_v.at[2 * p + t - b0]], rows[t], sems[t]).wait()
      _pack_pair(rows[b0], rows[b0 + 1], pks[b0 // 2])

      @pl.when(2 * p + 4 < n_ch)
      def _():
        for t in (b0, b0 + 1):
          pltpu.async_copy(
              table_hbm.at[idx_v.at[2 * p + 4 + t - b0]], rows[t], sems[t])

      pltpu.sync_copy(pks[b0 // 2], out_hbm.at[pl.ds(p * b + col0, CH)])

    def duo(q, _):
      handle_pair(2 * q, 0)
      handle_pair(2 * q + 1, 2)
      return ()

    lax.fori_loop(0, n_pairs // 2, duo, ())
    if n_pairs % 2:
      handle_pair(jnp.int32(n_pairs - 1), 0)

  return k(table, ids_t)


def _tc_main(x, g, wa, ba, wt, bt, rows_per_tile, tile0, n_tiles, l_seg):
  r = rows_per_tile
  b = (n_tiles * r) // l_seg

  def body(x_ref, g_ref, wa_ref, ba_ref, wt_ref, bt_ref, out_ref, m_s):
    @pl.when(pl.program_id(0) == 0)
    def _():
      m_s[...] = lax.dot_general(
          wa_ref[...], wt_ref[...], (((0,), (0,)), ((), ())),
          preferred_element_type=jnp.float32)

    xv = x_ref[...]
    gv2 = g_ref[...]
    glo = lax.bitcast_convert_type(gv2 << 16, jnp.float32)
    ghi = lax.bitcast_convert_type(gv2 & jnp.int32(-65536), jnp.float32)
    v = jnp.dot(ba_ref[...], wt_ref[...], preferred_element_type=jnp.float32)
    u = jnp.dot(bt_ref[...], wa_ref[...], preferred_element_type=jnp.float32)
    c = jnp.sum(ba_ref[...] * bt_ref[...])
    a = jnp.dot(xv, m_s[...], preferred_element_type=jnp.float32) + v
    ones = jnp.ones((1, D), dtype=jnp.float32)
    r2 = r // 2
    res_lo = lax.dot_general(
        ones, a[:r2] * glo, (((1,), (1,)), ((), ())),
        preferred_element_type=jnp.float32)
    res_hi = lax.dot_general(
        ones, a[r2:] * ghi, (((1,), (1,)), ((), ())),
        preferred_element_type=jnp.float32)
    z = lax.dot_general(
        u, xv, (((1,), (1,)), ((), ())),
        preferred_element_type=jnp.float32)
    row0 = res_lo + z[:, :r2] + c
    row1 = res_hi + z[:, r2:] + c
    out_ref[...] = jnp.concatenate([row0, row1], axis=0).reshape(2, 1, r2)

  out = pl.pallas_call(
      body,
      grid=(n_tiles,),
      in_specs=[
          pl.BlockSpec((r, D), lambda i: (tile0 + i, 0)),
          pl.BlockSpec((r // 2, D), lambda i: (i, 0)),
          pl.BlockSpec((D, D), lambda i: (0, 0)),
          pl.BlockSpec((1, D), lambda i: (0, 0)),
          pl.BlockSpec((D, D), lambda i: (0, 0)),
          pl.BlockSpec((1, D), lambda i: (0, 0)),
      ],
      out_specs=pl.BlockSpec((r // b, 1, b), lambda i: (i, 0, 0)),
      out_shape=jax.ShapeDtypeStruct((l_seg, 1, b), jnp.float32),
      scratch_shapes=[pltpu.VMEM((D, D), jnp.float32)],
  )(x, g, wa, ba, wt, bt)
  return out


def kernel(actor_emb, topic_ids, Wa, ba, table, Wt, bt, scale):
  b, l, d = actor_emb.shape
  bl = b * l

  wa_s = Wa * scale
  ba_s = (ba * scale).reshape(1, d)

  ids_t = topic_ids.T.astype(jnp.int32)
  x = actor_emb.transpose(1, 0, 2).reshape(bl, d)

  n_seg = 5
  l_seg = l // n_seg
  r = 8192
  nt_seg = l_seg * b // r
  bt_r = bt.reshape(1, d)
  outs = []
  for s in range(n_seg):
    ids_seg = lax.slice_in_dim(ids_t, s * l_seg, (s + 1) * l_seg, axis=0)
    g_seg = _sc_gather(table, ids_seg)
    outs.append(_tc_main(x, g_seg, wa_s, ba_s, Wt, bt_r, r,
                         s * nt_seg, nt_seg, l_seg))
  out = jnp.concatenate(outs, axis=0)
  return out.reshape(l, b).T


# speedup vs baseline: 1.1980x; 1.0117x over previous
import functools

import jax
import jax.numpy as jnp
from jax import lax
from jax.experimental import pallas as pl
from jax.experimental.pallas import tpu as pltpu
from jax.experimental.pallas import tpu_sc as plsc

D = 128
NC = 2
NS = 16
NW = NC * NS
CH = 128
NBUF = 4


def _sc_gather(table, ids_t):
  n_ch, b = ids_t.shape
  n_pairs = n_ch // 2
  total2 = n_pairs * b
  mesh = plsc.VectorSubcoreMesh(
      core_axis_name="c", subcore_axis_name="s", num_cores=NC, num_subcores=NS
  )

  def _pack_pair(rows_a, rows_b, pk):
    def rowbody(j, _):
      for grp in range(D // 16):
        cc = grp * 16
        ua = lax.bitcast_convert_type(rows_a[j, pl.ds(cc, 16)], jnp.int32)
        ub = lax.bitcast_convert_type(rows_b[j, pl.ds(cc, 16)], jnp.int32)
        ra = lax.shift_right_logical(ua + 32768, 16)
        rb = (ub + 32768) & jnp.int32(-65536)
        pk[j, pl.ds(cc, 16)] = ra | rb
      return ()

    lax.fori_loop(0, CH, rowbody, ())

  @functools.partial(
      pl.kernel,
      out_type=jax.ShapeDtypeStruct((total2, D), jnp.int32),
      mesh=mesh,
      scratch_types=[
          pltpu.VMEM((n_ch, CH), jnp.int32),
          [pltpu.VMEM((CH, D), jnp.float32) for _ in range(4)],
          [pltpu.VMEM((CH, D), jnp.int32) for _ in range(2)],
          [pltpu.SemaphoreType.DMA for _ in range(4)],
      ],
  )
  def k(table_hbm, ids_hbm, out_hbm, idx_v, rows, pks, sems):
    wid = lax.axis_index("s") * NC + lax.axis_index("c")
    col0 = wid * CH
    pltpu.sync_copy(ids_hbm.at[pl.ds(0, n_ch), pl.ds(col0, CH)], idx_v)
    for j in range(min(4, n_ch)):
      pltpu.async_copy(table_hbm.at[idx_v.at[j]], rows[j], sems[j])

    def handle_pair(p, b0):
      for t in (b0, b0 + 1):
        pltpu.make_async_copy(
            table_hbm.at[idx_v.at[2 * p + t - b0]], rows[t], sems[t]).wait()
      _pack_pair(rows[b0], rows[b0 + 1], pks[b0 // 2])

      @pl.when(2 * p + 4 < n_ch)
      def _():
        for t in (b0, b0 + 1):
          pltpu.async_copy(
              table_hbm.at[idx_v.at[2 * p + 4 + t - b0]], rows[t], sems[t])

      pltpu.sync_copy(pks[b0 // 2], out_hbm.at[pl.ds(p * b + col0, CH)])

    def duo(q, _):
      handle_pair(2 * q, 0)
      handle_pair(2 * q + 1, 2)
      return ()

    lax.fori_loop(0, n_pairs // 2, duo, ())
    if n_pairs % 2:
      handle_pair(jnp.int32(n_pairs - 1), 0)

  return k(table, ids_t)


def _tc_main(x, g, wa, ba, wt, bt, rows_per_tile, tile0, n_tiles, l_seg):
  r = rows_per_tile
  b = (n_tiles * r) // l_seg

  def body(x_ref, g_ref, wa_ref, ba_ref, wt_ref, bt_ref, out_ref, m_s):
    @pl.when(pl.program_id(0) == 0)
    def _():
      m_s[...] = lax.dot_general(
          wa_ref[...], wt_ref[...], (((0,), (0,)), ((), ())),
          preferred_element_type=jnp.float32)

    xv = x_ref[...]
    gv2 = g_ref[...]
    glo = lax.bitcast_convert_type(gv2 << 16, jnp.float32)
    ghi = lax.bitcast_convert_type(gv2 & jnp.int32(-65536), jnp.float32)
    v = jnp.dot(ba_ref[...], wt_ref[...], preferred_element_type=jnp.float32)
    u = jnp.dot(bt_ref[...], wa_ref[...], preferred_element_type=jnp.float32)
    c = jnp.sum(ba_ref[...] * bt_ref[...])
    a = jnp.dot(xv, m_s[...], preferred_element_type=jnp.float32) + v
    ones = jnp.ones((1, D), dtype=jnp.float32)
    r2 = r // 2
    res_lo = lax.dot_general(
        ones, a[:r2] * glo, (((1,), (1,)), ((), ())),
        preferred_element_type=jnp.float32)
    res_hi = lax.dot_general(
        ones, a[r2:] * ghi, (((1,), (1,)), ((), ())),
        preferred_element_type=jnp.float32)
    z = lax.dot_general(
        u, xv, (((1,), (1,)), ((), ())),
        preferred_element_type=jnp.float32)
    row0 = res_lo + z[:, :r2] + c
    row1 = res_hi + z[:, r2:] + c
    out_ref[...] = jnp.concatenate([row0, row1], axis=0).reshape(2, 1, r2)

  out = pl.pallas_call(
      body,
      grid=(n_tiles,),
      in_specs=[
          pl.BlockSpec((r, D), lambda i: (tile0 + i, 0)),
          pl.BlockSpec((r // 2, D), lambda i: (i, 0)),
          pl.BlockSpec((D, D), lambda i: (0, 0)),
          pl.BlockSpec((1, D), lambda i: (0, 0)),
          pl.BlockSpec((D, D), lambda i: (0, 0)),
          pl.BlockSpec((1, D), lambda i: (0, 0)),
      ],
      out_specs=pl.BlockSpec((r // b, 1, b), lambda i: (i, 0, 0)),
      out_shape=jax.ShapeDtypeStruct((l_seg, 1, b), jnp.float32),
      scratch_shapes=[pltpu.VMEM((D, D), jnp.float32)],
  )(x, g, wa, ba, wt, bt)
  return out


def kernel(actor_emb, topic_ids, Wa, ba, table, Wt, bt, scale):
  b, l, d = actor_emb.shape
  bl = b * l

  wa_s = Wa * scale
  ba_s = (ba * scale).reshape(1, d)

  ids_t = topic_ids.T.astype(jnp.int32)
  x = actor_emb.transpose(1, 0, 2).reshape(bl, d)

  segs = (2, 12, 12, 12, 12)
  r = 8192
  bt_r = bt.reshape(1, d)
  outs = []
  l0 = 0
  for l_seg in segs:
    ids_seg = lax.slice_in_dim(ids_t, l0, l0 + l_seg, axis=0)
    g_seg = _sc_gather(table, ids_seg)
    outs.append(_tc_main(x, g_seg, wa_s, ba_s, Wt, bt_r, r,
                         l0 * b // r, l_seg * b // r, l_seg))
    l0 += l_seg
  out = jnp.concatenate(outs, axis=0)
  return out.reshape(l, b).T
